# Initial kernel scaffold; baseline (speedup 1.0000x reference)
import jax
import jax.numpy as jnp
from jax.experimental import pallas as pl


def kernel(pos, x, edge_index, W1, W2):
    raise NotImplementedError("write your pallas kernel here")


# speedup vs baseline: 19.4316x; 19.4316x over previous
import functools

import numpy as np
import jax
import jax.numpy as jnp
from jax import lax
from jax.experimental import pallas as pl
from jax.experimental.pallas import tpu as pltpu
from jax.experimental.pallas import tpu_sc as plsc

_N = 10000
_E = 160000
_NW = 32
_EPT = 5120
_EPAD = _NW * _EPT
_CH = 128
_NCHUNK = _EPT // _CH
_NBLK = _CH // 16
_ISTEP = 11.0 / 2.0
_K = float(1.14136 * np.exp(2.0) * np.sqrt(2.0) / (16.0 * np.sqrt(128.0)))


def _tc_matmul_body(x_ref, w_ref, o_ref):
    o_ref[...] = jnp.dot(x_ref[...], w_ref[...],
                         preferred_element_type=jnp.float32)


def _tc_y(x, w2st):
    return pl.pallas_call(
        _tc_matmul_body,
        out_shape=jax.ShapeDtypeStruct((_N, 16), jnp.float32),
    )(x, w2st)


def _sc_edge_sum(posflat, src, dst, y, w1flat):
    mesh = plsc.VectorSubcoreMesh(core_axis_name="c", subcore_axis_name="s")

    @functools.partial(
        pl.kernel,
        mesh=mesh,
        out_type=jax.ShapeDtypeStruct((_NW, 16), jnp.float32),
        compiler_params=pltpu.CompilerParams(
            needs_layout_passes=False, use_tc_tiling_on_sc=False),
        scratch_types=[
            pltpu.VMEM((3 * _N,), jnp.float32),
            pltpu.VMEM((_EPT,), jnp.int32),
            pltpu.VMEM((_EPT,), jnp.int32),
            pltpu.VMEM((160,), jnp.float32),
            pltpu.VMEM((_CH, 16), jnp.float32),
            pltpu.VMEM((16,), jnp.float32),
            pltpu.SemaphoreType.DMA,
        ],
    )
    def k(pos_hbm, src_hbm, dst_hbm, y_hbm, w1_hbm, out_hbm,
          pos_v, src_v, dst_v, w1_v, ybuf_v, res_v, sem):
        cid = lax.axis_index("c")
        sid = lax.axis_index("s")
        wid = cid * 16 + sid
        base = wid * _EPT
        pltpu.sync_copy(pos_hbm, pos_v)
        pltpu.sync_copy(w1_hbm, w1_v)
        pltpu.sync_copy(src_hbm.at[pl.ds(base, _EPT)], src_v)
        pltpu.sync_copy(dst_hbm.at[pl.ds(base, _EPT)], dst_v)

        def chunk_body(c, tot):
            pltpu.async_copy(
                y_hbm.at[src_v.at[pl.ds(c * _CH, _CH)]], ybuf_v, sem
            ).wait()

            def blk(b, acc):
                off = c * _CH + b * 16
                s = src_v[pl.ds(off, 16)]
                d = dst_v[pl.ds(off, 16)]
                dx = plsc.load_gather(pos_v, [d]) - plsc.load_gather(pos_v, [s])
                dy = (plsc.load_gather(pos_v, [d + _N])
                      - plsc.load_gather(pos_v, [s + _N]))
                dz = (plsc.load_gather(pos_v, [d + 2 * _N])
                      - plsc.load_gather(pos_v, [s + 2 * _N]))
                r2 = dx * dx + dy * dy + dz * dz
                yb = lax.bitcast_convert_type(
                    jnp.int32(0x5F3759DF)
                    - (lax.bitcast_convert_type(r2, jnp.int32) >> 1),
                    jnp.float32)
                h2 = 0.5 * r2
                yb = yb * (1.5 - h2 * yb * yb)
                yb = yb * (1.5 - h2 * yb * yb)
                yb = yb * (1.5 - h2 * yb * yb)
                u = (r2 * yb) * _ISTEP
                f = u.astype(jnp.int32)
                fr = u - f.astype(jnp.float32)
                ta = 1.0 - fr * fr
                tb = fr * (2.0 - fr)
                ea = jnp.where((f >= 1) & (f <= 10), jnp.exp(-2.0 / ta), 0.0)
                eb = jnp.where((f <= 9) & (tb > 0.0), jnp.exp(-2.0 / tb), 0.0)
                ia = jnp.clip(f - 1, 0, 9) * 16
                ib = jnp.clip(f, 0, 9) * 16
                rows = lax.iota(jnp.int32, 16) + b * 16
                for j in range(16):
                    w1a = plsc.load_gather(w1_v, [ia + j])
                    w1b = plsc.load_gather(w1_v, [ib + j])
                    yj = plsc.load_gather(
                        ybuf_v, [rows, jnp.full((16,), j, jnp.int32)])
                    h = jnp.maximum(ea * w1a + eb * w1b, 0.0)
                    acc = acc + h * yj
                return acc

            return lax.fori_loop(0, _NBLK, blk, tot)

        tot = lax.fori_loop(0, _NCHUNK, chunk_body,
                            jnp.zeros((16,), jnp.float32))
        res_v[...] = tot
        pltpu.sync_copy(res_v, out_hbm.at[wid])

    return k(posflat, src, dst, y, w1flat)


def kernel(pos, x, edge_index, W1, W2):
    posflat = pos.T.reshape(-1)
    w2s = W2.reshape(16, 128, 4).sum(-1)
    y = _tc_y(x, w2s.T)
    padn = _EPAD - _E
    src = jnp.concatenate([edge_index[0],
                           jnp.zeros((padn,), jnp.int32)])
    dst = jnp.concatenate([edge_index[1],
                           jnp.zeros((padn,), jnp.int32)])
    w1flat = W1.reshape(-1)
    out = _sc_edge_sum(posflat, src, dst, y, w1flat)
    return jnp.stack([out.sum() * _K])


# speedup vs baseline: 23.9484x; 1.2324x over previous
import functools

import numpy as np
import jax
import jax.numpy as jnp
from jax import lax
from jax.experimental import pallas as pl
from jax.experimental.pallas import tpu as pltpu
from jax.experimental.pallas import tpu_sc as plsc

_N = 10000
_E = 160000
_NW = 32
_EPT = 5120
_EPAD = _NW * _EPT
_CH = 128
_NCHUNK = _EPT // _CH
_NBLK = _CH // 16
_ISTEP = 11.0 / 2.0
_K = float(1.14136 * np.exp(2.0) * np.sqrt(2.0) / (16.0 * np.sqrt(128.0)))


def _tc_matmul_body(x_ref, w_ref, o_ref):
    o_ref[...] = jnp.dot(x_ref[...], w_ref[...],
                         precision=jax.lax.Precision.HIGHEST,
                         preferred_element_type=jnp.float32)


def _tc_y(x, w2st):
    return pl.pallas_call(
        _tc_matmul_body,
        out_shape=jax.ShapeDtypeStruct((_N, 16), jnp.float32),
    )(x, w2st)


def _sc_edge_sum(posflat, src, dst, y, w1flat):
    mesh = plsc.VectorSubcoreMesh(core_axis_name="c", subcore_axis_name="s")

    @functools.partial(
        pl.kernel,
        mesh=mesh,
        out_type=jax.ShapeDtypeStruct((_NW, 16), jnp.float32),
        compiler_params=pltpu.CompilerParams(
            needs_layout_passes=False, use_tc_tiling_on_sc=False),
        scratch_types=[
            pltpu.VMEM((3 * _N,), jnp.float32),
            pltpu.VMEM((_EPT,), jnp.int32),
            pltpu.VMEM((_EPT,), jnp.int32),
            pltpu.VMEM((160,), jnp.float32),
            pltpu.VMEM((_CH, 16), jnp.float32),
            pltpu.VMEM((_CH, 16), jnp.float32),
            pltpu.VMEM((16,), jnp.float32),
            pltpu.SemaphoreType.DMA,
            pltpu.SemaphoreType.DMA,
            pltpu.SemaphoreType.DMA,
        ],
    )
    def k(pos_hbm, src_hbm, dst_hbm, y_hbm, w1_hbm, out_hbm,
          pos_v, src_v, dst_v, w1_v, ybuf0_v, ybuf1_v, res_v,
          sem0, sem1, semt):
        cid = lax.axis_index("c")
        sid = lax.axis_index("s")
        wid = cid * 16 + sid
        base = wid * _EPT
        pltpu.async_copy(src_hbm.at[pl.ds(base, _EPT)], src_v, semt)
        pltpu.async_copy(dst_hbm.at[pl.ds(base, _EPT)], dst_v, semt)
        pltpu.async_copy(pos_hbm, pos_v, semt)
        pltpu.async_copy(w1_hbm, w1_v, semt)
        pltpu.make_async_copy(src_hbm.at[pl.ds(base, _EPT)], src_v,
                              semt).wait()

        def issue(c, buf, sem):
            pltpu.async_copy(
                y_hbm.at[src_v.at[pl.ds(c * _CH, _CH)]], buf, sem)

        def drain(buf, sem):
            pltpu.make_async_copy(y_hbm.at[pl.ds(0, _CH)], buf, sem).wait()

        issue(0, ybuf0_v, sem0)
        pltpu.make_async_copy(dst_hbm.at[pl.ds(base, _EPT)], dst_v,
                              semt).wait()
        pltpu.make_async_copy(pos_hbm, pos_v, semt).wait()
        pltpu.make_async_copy(w1_hbm, w1_v, semt).wait()

        def compute_chunk(c, ybuf_v, tot):
            def blk(b, carry):
                tot, comp = carry
                off = c * _CH + b * 16
                s = src_v[pl.ds(off, 16)] * 3
                d = dst_v[pl.ds(off, 16)] * 3
                dx = plsc.load_gather(pos_v, [d]) - plsc.load_gather(pos_v, [s])
                dy = (plsc.load_gather(pos_v, [d + 1])
                      - plsc.load_gather(pos_v, [s + 1]))
                dz = (plsc.load_gather(pos_v, [d + 2])
                      - plsc.load_gather(pos_v, [s + 2]))
                r2 = dx * dx + dy * dy + dz * dz
                yb = lax.bitcast_convert_type(
                    jnp.int32(0x5F3759DF)
                    - (lax.bitcast_convert_type(r2, jnp.int32) >> 1),
                    jnp.float32)
                h2 = 0.5 * r2
                yb = yb * (1.5 - h2 * yb * yb)
                yb = yb * (1.5 - h2 * yb * yb)
                yb = yb * (1.5 - h2 * yb * yb)
                u = (r2 * yb) * _ISTEP
                f = u.astype(jnp.int32)
                fr = u - f.astype(jnp.float32)
                ta = 1.0 - fr * fr
                tb = fr * (2.0 - fr)
                ea = jnp.where((f >= 1) & (f <= 10), jnp.exp(-2.0 / ta), 0.0)
                eb = jnp.where((f <= 9) & (tb > 0.0), jnp.exp(-2.0 / tb), 0.0)
                ia = jnp.clip(f - 1, 0, 9) * 16
                ib = jnp.clip(f, 0, 9) * 16
                rows = lax.iota(jnp.int32, 16) + b * 16
                bacc = jnp.zeros((16,), jnp.float32)
                for j in range(16):
                    w1a = plsc.load_gather(w1_v, [ia + j])
                    w1b = plsc.load_gather(w1_v, [ib + j])
                    yj = plsc.load_gather(
                        ybuf_v, [rows, jnp.full((16,), j, jnp.int32)])
                    h = jnp.maximum(ea * w1a + eb * w1b, 0.0)
                    bacc = bacc + h * yj
                yk = bacc - comp
                t = tot + yk
                comp = (t - tot) - yk
                return t, comp

            return lax.fori_loop(0, _NBLK, blk, tot)

        def pair(p, tot):
            c0 = 2 * p
            issue(c0 + 1, ybuf1_v, sem1)
            drain(ybuf0_v, sem0)
            tot = compute_chunk(c0, ybuf0_v, tot)
            issue(jnp.minimum(c0 + 2, _NCHUNK - 1), ybuf0_v, sem0)
            drain(ybuf1_v, sem1)
            return compute_chunk(c0 + 1, ybuf1_v, tot)

        tot, _ = lax.fori_loop(
            0, _NCHUNK // 2, pair,
            (jnp.zeros((16,), jnp.float32), jnp.zeros((16,), jnp.float32)))
        drain(ybuf0_v, sem0)
        res_v[...] = tot
        pltpu.sync_copy(res_v, out_hbm.at[wid])

    return k(posflat, src, dst, y, w1flat)


def kernel(pos, x, edge_index, W1, W2):
    posflat = pos.reshape(-1)
    w2s = W2.reshape(16, 128, 4).sum(-1)
    y = _tc_y(x, w2s.T)
    padn = _EPAD - _E
    src = jnp.concatenate([edge_index[0],
                           jnp.zeros((padn,), jnp.int32)])
    dst = jnp.concatenate([edge_index[1],
                           jnp.zeros((padn,), jnp.int32)])
    w1flat = W1.reshape(-1)
    out = _sc_edge_sum(posflat, src, dst, y, w1flat)
    return jnp.stack([out.sum() * _K])
